# baseline (speedup 1.0000x reference)
import functools

import jax
import jax.numpy as jnp
from jax import lax
from jax.experimental import pallas as pl
from jax.experimental.pallas import tpu as pltpu
from jax.experimental.pallas import tpu_sc as plsc

_B = 16
_N = 512
_E = 1536
_T = _N + _E
_D = 1024
_K2 = 256
_VOCAB = 512

_NC = 2
_NS = 16
_NW = _NC * _NS
_L = 16

_CH = 16

_GROUP_SIZES = (2, 4, 4, 4, 2)
assert sum(_GROUP_SIZES) == _B


def _make_sc_body(b0, bg):
    def body(node_hbm, edge_hbm, emb_hbm, out_hbm,
             table_v, in0, in1, out0, out1,
             sin0, sin1, sout0, sout1):
        rpt = bg * _T // _NW
        tpb = _NW // bg
        c = lax.axis_index("c")
        s = lax.axis_index("s")
        wid = s * _NC + c
        b_local = wid // tpb
        t0 = (wid % tpb) * rpt
        b = b0 + b_local
        dst0 = wid * rpt

        pltpu.sync_copy(emb_hbm, table_v)

        ins = (in0, in1)
        outs = (out0, out1)
        sins = (sin0, sin1)
        souts = (sout0, sout1)

        himask = jnp.int32(-65536)

        def gather_chunk(iv, ov):
            def row_body(i, carry):
                @plsc.parallel_loop(0, _D // 2, _L, unroll=4)
                def _(c):
                    c = pl.multiple_of(c, _L)
                    a = plsc.bitcast(plsc.load_gather(
                        table_v, [iv[i, pl.ds(c, _L)]]), jnp.int32)
                    b = plsc.bitcast(plsc.load_gather(
                        table_v, [iv[i, pl.ds(c + _D // 2, _L)]]), jnp.int32)
                    ov[i, pl.ds(c, _L)] = (
                        lax.shift_right_logical(a, 16) | (b & himask))
                return carry
            lax.fori_loop(0, _CH, row_body, 0, unroll=False)

        def run_segment(src_hbm, src_row0):
            nch = rpt // _CH

            def start_in(ci, buf):
                pltpu.make_async_copy(
                    src_hbm.at[pl.ds(src_row0 + ci * _CH, _CH), :],
                    ins[buf], sins[buf]).start()

            def wait_in(buf):
                pltpu.make_async_copy(
                    src_hbm.at[pl.ds(src_row0, _CH), :],
                    ins[buf], sins[buf]).wait()

            def start_out(ci, buf):
                pltpu.make_async_copy(
                    outs[buf],
                    out_hbm.at[pl.ds(dst0 + ci * _CH, _CH), :],
                    souts[buf]).start()

            def wait_out(buf):
                pltpu.make_async_copy(
                    outs[buf],
                    out_hbm.at[pl.ds(dst0, _CH), :],
                    souts[buf]).wait()

            start_in(0, 0)

            def loop_body(j, carry):
                ca = j * 2

                start_in(ca + 1, 1)
                wait_in(0)

                @pl.when(j > 0)
                def _():
                    wait_out(0)

                gather_chunk(ins[0], outs[0])
                start_out(ca, 0)

                cb = ca + 1

                @pl.when(cb + 1 < nch)
                def _():
                    start_in(cb + 1, 0)

                wait_in(1)

                @pl.when(j > 0)
                def _():
                    wait_out(1)

                gather_chunk(ins[1], outs[1])
                start_out(cb, 1)
                return carry

            lax.fori_loop(0, nch // 2, loop_body, 0, unroll=False)
            wait_out(0)
            wait_out(1)

        @pl.when(t0 < _N)
        def _():
            run_segment(node_hbm, b * _N + t0)

        @pl.when(t0 >= _N)
        def _():
            run_segment(edge_hbm, b * _E + (t0 - _N))

    return body


@functools.lru_cache(maxsize=None)
def _sc_gather(b0, bg):
    return pl.kernel(
        _make_sc_body(b0, bg),
        out_type=jax.ShapeDtypeStruct((bg * _T, _D // 2), jnp.int32),
        mesh=plsc.VectorSubcoreMesh(core_axis_name="c", subcore_axis_name="s",
                                    num_cores=_NC, num_subcores=_NS),
        scratch_types=[
            pltpu.VMEM((_VOCAB,), jnp.float32),
            pltpu.VMEM((_CH, _D), jnp.int32),
            pltpu.VMEM((_CH, _D), jnp.int32),
            pltpu.VMEM((_CH, _D // 2), jnp.int32),
            pltpu.VMEM((_CH, _D // 2), jnp.int32),
            pltpu.SemaphoreType.DMA,
            pltpu.SemaphoreType.DMA,
            pltpu.SemaphoreType.DMA,
            pltpu.SemaphoreType.DMA,
        ],
        compiler_params=pltpu.CompilerParams(needs_layout_passes=False),
    )


_R = 512


def _tc_compute(feat_ref, lap_ref, pi_ref, mask_ref, w_ref, ord_ref, out_ref):
    mm = lax.dot_general(lap_ref[...], w_ref[...],
                         (((1,), (0,)), ((), ())),
                         preferred_element_type=jnp.float32)
    pi = pi_ref[...]
    order = pi[:, 0:1] == pi[:, 1:2]
    ordvec = jnp.where(order, ord_ref[1:2, :], ord_ref[0:1, :])
    rest = mm + ordvec
    w = feat_ref[...]
    left = lax.bitcast_convert_type(lax.shift_left(w, 16), jnp.float32)
    right = lax.bitcast_convert_type(w & jnp.int32(-65536), jnp.float32)
    keep = mask_ref[...] == 0
    half = _D // 2
    out_ref[:, 0:half] = jnp.where(
        keep, left + rest[:, 0:half], jnp.float32(0.0))
    out_ref[:, half:_D] = jnp.where(
        keep, right + rest[:, half:_D], jnp.float32(0.0))


def _tc_body_first(feat_ref, lap_ref, pi_ref, mask_ref, w_ref, ord_ref, out_ref):
    _tc_compute(feat_ref, lap_ref, pi_ref, mask_ref, w_ref, ord_ref, out_ref)


def _tc_body_acc(acc_ref, feat_ref, lap_ref, pi_ref, mask_ref, w_ref, ord_ref,
                 out_ref):
    del acc_ref
    _tc_compute(feat_ref, lap_ref, pi_ref, mask_ref, w_ref, ord_ref, out_ref)


def _tc_stage(row0, bg, acc, feat_g, lap2, pi2, mask2, w, ordt):
    bpg = bg * _T // _R
    blk0 = row0 // _R

    def off(i):
        return (i + blk0, 0)

    common_specs = [
        pl.BlockSpec((_R, _D // 2), lambda i: (i, 0)),
        pl.BlockSpec((_R, _K2), off),
        pl.BlockSpec((_R, 2), off),
        pl.BlockSpec((_R, 1), off),
        pl.BlockSpec((_K2, _D), lambda i: (0, 0)),
        pl.BlockSpec((2, _D), lambda i: (0, 0)),
    ]
    out_spec = pl.BlockSpec((_R, _D), off)
    out_shape = jax.ShapeDtypeStruct((_B * _T, _D), jnp.float32)
    if acc is None:
        return pl.pallas_call(
            _tc_body_first,
            grid=(bpg,),
            in_specs=common_specs,
            out_specs=out_spec,
            out_shape=out_shape,
        )(feat_g, lap2, pi2, mask2, w, ordt)
    return pl.pallas_call(
        _tc_body_acc,
        grid=(bpg,),
        in_specs=[pl.BlockSpec(memory_space=pl.ANY)] + common_specs,
        out_specs=out_spec,
        out_shape=out_shape,
        input_output_aliases={0: 0},
    )(acc, feat_g, lap2, pi2, mask2, w, ordt)


def kernel(edge_index, edge_data, node_data, node_num, edge_num, lap_eigvec,
           padded_index, padding_mask, padded_node_mask, padded_edge_mask,
           emb_table, W_lap, order_table):
    emb_flat = emb_table.reshape(-1)

    lap2 = lap_eigvec.reshape(_B * _T, _K2)
    pi2 = padded_index.reshape(_B * _T, 2)
    mask2 = padding_mask.reshape(_B * _T, 1).astype(jnp.int32)

    starts = [sum(_GROUP_SIZES[:i]) for i in range(len(_GROUP_SIZES))]
    feats = [_sc_gather(b0, bg)(node_data, edge_data, emb_flat)
             for b0, bg in zip(starts, _GROUP_SIZES)]
    out = None
    for b0, bg, feat_g in zip(starts, _GROUP_SIZES, feats):
        out = _tc_stage(b0 * _T, bg, out, feat_g,
                        lap2, pi2, mask2, W_lap, order_table)
    return out.reshape(_B, _T, _D)


# speedup vs baseline: 1.0184x; 1.0184x over previous
import functools

import jax
import jax.numpy as jnp
from jax import lax
from jax.experimental import pallas as pl
from jax.experimental.pallas import tpu as pltpu
from jax.experimental.pallas import tpu_sc as plsc

_B = 16
_N = 512
_E = 1536
_T = _N + _E
_D = 1024
_K2 = 256
_VOCAB = 512

_NC = 2
_NS = 16
_NW = _NC * _NS
_L = 16

_CH = 16

_GROUP_SIZES = (4, 4, 4, 4)
assert sum(_GROUP_SIZES) == _B


def _make_sc_body(b0, bg):
    def body(node_hbm, edge_hbm, emb_hbm, out_hbm,
             table_v, in0, in1, out0, out1,
             sin0, sin1, sout0, sout1):
        rpt = bg * _T // _NW
        tpb = _NW // bg
        c = lax.axis_index("c")
        s = lax.axis_index("s")
        wid = s * _NC + c
        b_local = wid // tpb
        t0 = (wid % tpb) * rpt
        b = b0 + b_local
        dst0 = wid * rpt

        pltpu.sync_copy(emb_hbm, table_v)

        ins = (in0, in1)
        outs = (out0, out1)
        sins = (sin0, sin1)
        souts = (sout0, sout1)

        himask = jnp.int32(-65536)

        def gather_chunk(iv, ov):
            def row_body(i, carry):
                @plsc.parallel_loop(0, _D // 2, _L, unroll=4)
                def _(c):
                    c = pl.multiple_of(c, _L)
                    a = plsc.bitcast(plsc.load_gather(
                        table_v, [iv[i, pl.ds(c, _L)]]), jnp.int32)
                    b = plsc.bitcast(plsc.load_gather(
                        table_v, [iv[i, pl.ds(c + _D // 2, _L)]]), jnp.int32)
                    ov[i, pl.ds(c, _L)] = (
                        lax.shift_right_logical(a, 16) | (b & himask))
                return carry
            lax.fori_loop(0, _CH, row_body, 0, unroll=False)

        def run_segment(src_hbm, src_row0):
            nch = rpt // _CH

            def start_in(ci, buf):
                pltpu.make_async_copy(
                    src_hbm.at[pl.ds(src_row0 + ci * _CH, _CH), :],
                    ins[buf], sins[buf]).start()

            def wait_in(buf):
                pltpu.make_async_copy(
                    src_hbm.at[pl.ds(src_row0, _CH), :],
                    ins[buf], sins[buf]).wait()

            def start_out(ci, buf):
                pltpu.make_async_copy(
                    outs[buf],
                    out_hbm.at[pl.ds(dst0 + ci * _CH, _CH), :],
                    souts[buf]).start()

            def wait_out(buf):
                pltpu.make_async_copy(
                    outs[buf],
                    out_hbm.at[pl.ds(dst0, _CH), :],
                    souts[buf]).wait()

            start_in(0, 0)

            def loop_body(j, carry):
                ca = j * 2

                start_in(ca + 1, 1)
                wait_in(0)

                @pl.when(j > 0)
                def _():
                    wait_out(0)

                gather_chunk(ins[0], outs[0])
                start_out(ca, 0)

                cb = ca + 1

                @pl.when(cb + 1 < nch)
                def _():
                    start_in(cb + 1, 0)

                wait_in(1)

                @pl.when(j > 0)
                def _():
                    wait_out(1)

                gather_chunk(ins[1], outs[1])
                start_out(cb, 1)
                return carry

            lax.fori_loop(0, nch // 2, loop_body, 0, unroll=False)
            wait_out(0)
            wait_out(1)

        @pl.when(t0 < _N)
        def _():
            run_segment(node_hbm, b * _N + t0)

        @pl.when(t0 >= _N)
        def _():
            run_segment(edge_hbm, b * _E + (t0 - _N))

    return body


@functools.lru_cache(maxsize=None)
def _sc_gather(b0, bg):
    return pl.kernel(
        _make_sc_body(b0, bg),
        out_type=jax.ShapeDtypeStruct((bg * _T, _D // 2), jnp.int32),
        mesh=plsc.VectorSubcoreMesh(core_axis_name="c", subcore_axis_name="s",
                                    num_cores=_NC, num_subcores=_NS),
        scratch_types=[
            pltpu.VMEM((_VOCAB,), jnp.float32),
            pltpu.VMEM((_CH, _D), jnp.int32),
            pltpu.VMEM((_CH, _D), jnp.int32),
            pltpu.VMEM((_CH, _D // 2), jnp.int32),
            pltpu.VMEM((_CH, _D // 2), jnp.int32),
            pltpu.SemaphoreType.DMA,
            pltpu.SemaphoreType.DMA,
            pltpu.SemaphoreType.DMA,
            pltpu.SemaphoreType.DMA,
        ],
        compiler_params=pltpu.CompilerParams(needs_layout_passes=False),
    )


_R = 1024


def _tc_compute(feat_ref, lap_ref, pi_ref, mask_ref, w_ref, ord_ref, out_ref):
    mm = lax.dot_general(lap_ref[...], w_ref[...],
                         (((1,), (0,)), ((), ())),
                         preferred_element_type=jnp.float32)
    pi = pi_ref[...]
    order = pi[:, 0:1] == pi[:, 1:2]
    ordvec = jnp.where(order, ord_ref[1:2, :], ord_ref[0:1, :])
    rest = mm + ordvec
    w = feat_ref[...]
    left = lax.bitcast_convert_type(lax.shift_left(w, 16), jnp.float32)
    right = lax.bitcast_convert_type(w & jnp.int32(-65536), jnp.float32)
    keep = mask_ref[...] == 0
    half = _D // 2
    out_ref[:, 0:half] = jnp.where(
        keep, left + rest[:, 0:half], jnp.float32(0.0))
    out_ref[:, half:_D] = jnp.where(
        keep, right + rest[:, half:_D], jnp.float32(0.0))


def _tc_body_first(feat_ref, lap_ref, pi_ref, mask_ref, w_ref, ord_ref, out_ref):
    _tc_compute(feat_ref, lap_ref, pi_ref, mask_ref, w_ref, ord_ref, out_ref)


def _tc_body_acc(acc_ref, feat_ref, lap_ref, pi_ref, mask_ref, w_ref, ord_ref,
                 out_ref):
    del acc_ref
    _tc_compute(feat_ref, lap_ref, pi_ref, mask_ref, w_ref, ord_ref, out_ref)


def _tc_stage(row0, bg, acc, feat_g, lap2, pi2, mask2, w, ordt):
    bpg = bg * _T // _R
    blk0 = row0 // _R

    def off(i):
        return (i + blk0, 0)

    common_specs = [
        pl.BlockSpec((_R, _D // 2), lambda i: (i, 0)),
        pl.BlockSpec((_R, _K2), off),
        pl.BlockSpec((_R, 2), off),
        pl.BlockSpec((_R, 1), off),
        pl.BlockSpec((_K2, _D), lambda i: (0, 0)),
        pl.BlockSpec((2, _D), lambda i: (0, 0)),
    ]
    out_spec = pl.BlockSpec((_R, _D), off)
    out_shape = jax.ShapeDtypeStruct((_B * _T, _D), jnp.float32)
    if acc is None:
        return pl.pallas_call(
            _tc_body_first,
            grid=(bpg,),
            in_specs=common_specs,
            out_specs=out_spec,
            out_shape=out_shape,
        )(feat_g, lap2, pi2, mask2, w, ordt)
    return pl.pallas_call(
        _tc_body_acc,
        grid=(bpg,),
        in_specs=[pl.BlockSpec(memory_space=pl.ANY)] + common_specs,
        out_specs=out_spec,
        out_shape=out_shape,
        input_output_aliases={0: 0},
    )(acc, feat_g, lap2, pi2, mask2, w, ordt)


def kernel(edge_index, edge_data, node_data, node_num, edge_num, lap_eigvec,
           padded_index, padding_mask, padded_node_mask, padded_edge_mask,
           emb_table, W_lap, order_table):
    emb_flat = emb_table.reshape(-1)

    lap2 = lap_eigvec.reshape(_B * _T, _K2)
    pi2 = padded_index.reshape(_B * _T, 2)
    mask2 = padding_mask.reshape(_B * _T, 1).astype(jnp.int32)

    starts = [sum(_GROUP_SIZES[:i]) for i in range(len(_GROUP_SIZES))]
    feats = [_sc_gather(b0, bg)(node_data, edge_data, emb_flat)
             for b0, bg in zip(starts, _GROUP_SIZES)]
    out = None
    for b0, bg, feat_g in zip(starts, _GROUP_SIZES, feats):
        out = _tc_stage(b0 * _T, bg, out, feat_g,
                        lap2, pi2, mask2, W_lap, order_table)
    return out.reshape(_B, _T, _D)
